# baseline (speedup 1.0000x reference)
import functools

import jax
import jax.numpy as jnp
from jax import lax
from jax.experimental import pallas as pl
from jax.experimental.pallas import tpu as pltpu
from jax.experimental.pallas import tpu_sc as plsc

N = 10000
NP = 10240
E = 160000
D = 256
H = 128
NC = 2
NS = 16
L = 16
NW = NC * NS

ET = E // NS
CE = 80
NCHUNK = ET // CE

STRIPE = NP // NS

_mesh = plsc.VectorSubcoreMesh(core_axis_name="c", subcore_axis_name="s",
                               num_cores=NC, num_subcores=NS)


def _zero_rows(buf, nrows):
    def _body(i, _):
        for j in range(H // L):
            buf[i, pl.ds(j * L, L)] = jnp.zeros((L,), jnp.float32)
        return _
    lax.fori_loop(0, nrows, _body, None)


_DEG_C = 40
_DEG_PER_TILE = (E // NC) // NS
_DEG_NCHUNK = _DEG_PER_TILE // _DEG_C
_DEG_G = 5


@functools.partial(
    pl.kernel,
    out_type=jax.ShapeDtypeStruct((NC * NP, H), jnp.float32),
    mesh=_mesh,
    scratch_types=[
        pltpu.VMEM((CE, H), jnp.float32),
        pltpu.VMEM((_DEG_NCHUNK, _DEG_C), jnp.int32),
        pltpu.VMEM_SHARED((NP, H), jnp.float32),
        pltpu.SemaphoreType.DMA,
    ],
)
def _deg_kernel(dstd_hbm, deg_out, zob, didx, hist_sh, sem):
    c = lax.axis_index("c")
    s = lax.axis_index("s")
    w = c * NS + s

    _zero_rows(zob, CE)
    row0 = s * STRIPE
    for j in range(STRIPE // CE):
        pltpu.sync_copy(zob, hist_sh.at[pl.ds(row0 + j * CE, CE)])

    pltpu.sync_copy(dstd_hbm.at[w], didx)

    def _ones_body(i, _):
        for j in range(H // L):
            zob[i, pl.ds(j * L, L)] = jnp.ones((L,), jnp.float32)
        return _
    lax.fori_loop(0, _DEG_C, _ones_body, None)
    plsc.subcore_barrier()

    ones_rows = zob.at[pl.ds(0, _DEG_C)]

    def _grp_body(g, _):
        for j in range(_DEG_G):
            k = g * _DEG_G + j
            pltpu.async_copy(ones_rows, hist_sh.at[didx.at[k]], sem,
                             add=True)
        for j in range(_DEG_G):
            pltpu.make_async_copy(ones_rows, hist_sh.at[didx.at[0]],
                                  sem).wait()
        return _
    lax.fori_loop(0, _DEG_NCHUNK // _DEG_G, _grp_body, None)

    plsc.subcore_barrier()
    pltpu.sync_copy(hist_sh.at[pl.ds(row0, STRIPE)],
                    deg_out.at[pl.ds(c * NP + row0, STRIPE)])


@functools.partial(
    pl.kernel,
    out_type=jax.ShapeDtypeStruct((NC * NP, H), jnp.float32),
    mesh=_mesh,
    scratch_types=[
        pltpu.VMEM((ET,), jnp.int32),
        pltpu.VMEM((NCHUNK, CE), jnp.int32),
        pltpu.VMEM((CE, H), jnp.float32),
        pltpu.VMEM((CE, H), jnp.float32),
        pltpu.VMEM_SHARED((NP, H), jnp.float32),
        pltpu.SemaphoreType.DMA,
        pltpu.SemaphoreType.DMA,
        pltpu.SemaphoreType.DMA,
        pltpu.SemaphoreType.DMA,
    ],
)
def _agg_kernel(y_hbm, srca_hbm, dstu_hbm, z_out, idx_s, idx_d,
                rows_a, rows_b, z_sh, sem_a, sem_b, ssem_a, ssem_b):
    c = lax.axis_index("c")
    s = lax.axis_index("s")
    w = c * NS + s

    _zero_rows(rows_a, CE)
    row0 = s * STRIPE
    for j in range(STRIPE // CE):
        pltpu.sync_copy(rows_a, z_sh.at[pl.ds(row0 + j * CE, CE)])

    pltpu.sync_copy(srca_hbm.at[w], idx_s)
    pltpu.sync_copy(dstu_hbm.at[s], idx_d)
    plsc.subcore_barrier()

    def _gather(k, buf, sem):
        pltpu.async_copy(y_hbm.at[idx_s.at[pl.ds(k * CE, CE)]], buf, sem)

    def _wait(buf, sem):
        pltpu.make_async_copy(y_hbm.at[pl.ds(0, CE)], buf, sem).wait()

    def _wait_scat(buf, ssem):
        pltpu.make_async_copy(buf, z_sh.at[idx_d.at[0]], ssem).wait()

    def _path(k, buf, gsem, ssem, obuf, ogsem, ossem):
        @pl.when(k >= 2)
        def _():
            _wait_scat(buf, ssem)
        _gather(k, buf, gsem)

        @pl.when(k >= 1)
        def _():
            _wait(obuf, ogsem)
            pltpu.async_copy(obuf, z_sh.at[idx_d.at[k - 1]], ossem,
                             add=True)

    def _chunk_body(k, _):
        @pl.when(k % 2 == 0)
        def _():
            _path(k, rows_a, sem_a, ssem_a, rows_b, sem_b, ssem_b)

        @pl.when(k % 2 == 1)
        def _():
            _path(k, rows_b, sem_b, ssem_b, rows_a, sem_a, ssem_a)
        return _
    lax.fori_loop(0, NCHUNK, _chunk_body, None)

    _wait(rows_a, sem_a)
    pltpu.sync_copy(rows_a, z_sh.at[idx_d.at[NCHUNK - 1]], add=True)
    _wait_scat(rows_b, ssem_b)

    plsc.subcore_barrier()
    pltpu.sync_copy(z_sh.at[pl.ds(row0, STRIPE)],
                    z_out.at[pl.ds(c * NP + row0, STRIPE)])


@functools.partial(
    pl.kernel,
    out_type=jax.ShapeDtypeStruct((E, D), jnp.float32),
    mesh=_mesh,
    scratch_types=[
        pltpu.VMEM((NCHUNK, CE), jnp.int32),
        pltpu.VMEM((NCHUNK, CE), jnp.int32),
        pltpu.VMEM((CE, H), jnp.float32),
        pltpu.VMEM((CE, H), jnp.float32),
        pltpu.SemaphoreType.DMA,
        pltpu.SemaphoreType.DMA,
        pltpu.SemaphoreType.DMA,
        pltpu.SemaphoreType.DMA,
    ],
)
def _edge_kernel(h_hbm, srca_hbm, dsta_hbm, out_hbm, idx_s, idx_d,
                 buf_a, buf_b, gsem_a, gsem_b, wsem_a, wsem_b):
    c = lax.axis_index("c")
    s = lax.axis_index("s")
    w = c * NS + s
    base = s * ET
    col0 = c * H

    pltpu.sync_copy(srca_hbm.at[w], idx_s)
    pltpu.sync_copy(dsta_hbm.at[w], idx_d)

    def _wait_write(buf, wsem):
        pltpu.make_async_copy(
            buf, out_hbm.at[pl.ds(base, CE), pl.ds(col0, H)], wsem).wait()

    def _wait_gather(buf, gsem):
        pltpu.make_async_copy(h_hbm.at[pl.ds(0, CE)], buf, gsem).wait()

    def _gsrc(k, buf, gsem):
        pltpu.async_copy(h_hbm.at[idx_s.at[k]], buf, gsem)

    _gsrc(0, buf_a, gsem_a)

    def _path(k, buf, gsem, wsem, obuf, ogsem, owsem):
        _wait_gather(buf, gsem)
        pltpu.async_copy(h_hbm.at[idx_d.at[k]], buf, gsem, add=True)

        @pl.when(k < NCHUNK - 1)
        def _():
            @pl.when(k >= 1)
            def _():
                _wait_write(obuf, owsem)
            _gsrc(k + 1, obuf, ogsem)

        _wait_gather(buf, gsem)
        off = pl.multiple_of(base + k * CE, 8)
        pltpu.async_copy(buf, out_hbm.at[pl.ds(off, CE), pl.ds(col0, H)],
                         wsem)

    def _chunk_body(k, _):
        @pl.when(k % 2 == 0)
        def _():
            _path(k, buf_a, gsem_a, wsem_a, buf_b, gsem_b, wsem_b)

        @pl.when(k % 2 == 1)
        def _():
            _path(k, buf_b, gsem_b, wsem_b, buf_a, gsem_a, wsem_a)
        return _
    lax.fori_loop(0, NCHUNK, _chunk_body, None)

    _wait_write(buf_a, wsem_a)
    _wait_write(buf_b, wsem_b)


_BR = 640


def _dinv_block(deg_a, deg_b):
    return lax.rsqrt(deg_a[:, 0:1] + deg_b[:, 0:1] + 1.0)


def _tc_a_body(x_ref, w_ref, dega_ref, degb_ref, y_ref):
    dinv = _dinv_block(dega_ref[...], degb_ref[...])
    xw = jnp.dot(x_ref[...], w_ref[...], preferred_element_type=jnp.float32)
    y = xw * dinv
    y_ref[0] = y[:, :H]
    y_ref[1] = y[:, H:]


def _tc_b_body(zlo_ref, zhi_ref, ylo_ref, yhi_ref, dega_ref, degb_ref,
               b1_ref, w2_ref, y2_ref):
    dinv = _dinv_block(dega_ref[...], degb_ref[...])
    h_lo = jnp.maximum((zlo_ref[...] + ylo_ref[...]) * dinv
                       + b1_ref[0:1, :H], 0.0)
    h_hi = jnp.maximum((zhi_ref[...] + yhi_ref[...]) * dinv
                       + b1_ref[0:1, H:], 0.0)
    xw2 = (jnp.dot(h_lo, w2_ref[:H, :], preferred_element_type=jnp.float32)
           + jnp.dot(h_hi, w2_ref[H:, :], preferred_element_type=jnp.float32))
    y2 = xw2 * dinv
    y2_ref[0] = y2[:, :H]
    y2_ref[1] = y2[:, H:]


def _tc_c_body(zlo_ref, zhi_ref, ylo_ref, yhi_ref, dega_ref, degb_ref,
               b2_ref, h_ref):
    dinv = _dinv_block(dega_ref[...], degb_ref[...])
    h_ref[0] = ((zlo_ref[...] + ylo_ref[...]) * dinv + b2_ref[0:1, :H]) * 0.5
    h_ref[1] = ((zhi_ref[...] + yhi_ref[...]) * dinv + b2_ref[0:1, H:]) * 0.5


_row_spec = pl.BlockSpec((_BR, D), lambda i: (i, 0))
_half_lo_spec = pl.BlockSpec((_BR, H), lambda i: (i, 0))
_half_hi_spec = pl.BlockSpec((_BR, H), lambda i: (NP // _BR + i, 0))
_w_spec = pl.BlockSpec((D, D), lambda i: (0, 0))
_b_spec = pl.BlockSpec((1, D), lambda i: (0, 0))
_split_out_spec = pl.BlockSpec((NC, _BR, H), lambda i: (0, i, 0))
_split_out_shape = jax.ShapeDtypeStruct((NC, NP, H), jnp.float32)

_tc_a = pl.pallas_call(
    _tc_a_body,
    grid=(NP // _BR,),
    in_specs=[_row_spec, _w_spec, _half_lo_spec, _half_hi_spec],
    out_specs=_split_out_spec,
    out_shape=_split_out_shape,
)

_tc_b = pl.pallas_call(
    _tc_b_body,
    grid=(NP // _BR,),
    in_specs=[_half_lo_spec, _half_hi_spec, _half_lo_spec, _half_hi_spec,
              _half_lo_spec, _half_hi_spec, _b_spec, _w_spec],
    out_specs=_split_out_spec,
    out_shape=_split_out_shape,
)

_tc_c = pl.pallas_call(
    _tc_c_body,
    grid=(NP // _BR,),
    in_specs=[_half_lo_spec, _half_hi_spec, _half_lo_spec, _half_hi_spec,
              _half_lo_spec, _half_hi_spec, _b_spec],
    out_specs=_split_out_spec,
    out_shape=_split_out_shape,
)


def kernel(x, edge_index, W1, b1, W2, b2):
    ei = edge_index.astype(jnp.int32)
    src = ei[0]
    dst = ei[1]
    b1r = b1.reshape(1, D)
    b2r = b2.reshape(1, D)
    x_pad = jnp.concatenate(
        [x, jnp.zeros((NP - N, D), jnp.float32)], axis=0)

    src_t = src.reshape(NS, NCHUNK, CE)
    dst_t = dst.reshape(NS, NCHUNK, CE)
    src_a = jnp.concatenate([src_t, src_t + NP]).reshape(NW, NCHUNK, CE)
    dst_a = jnp.concatenate([dst_t, dst_t + NP]).reshape(NW, NCHUNK, CE)
    dst_d = dst.reshape(NW, _DEG_NCHUNK, _DEG_C)
    src_r = src.reshape(NS, ET)
    src_aa = jnp.concatenate([src_r, src_r + NP]).reshape(NW, ET)

    deg = _deg_kernel(dst_d)

    y1 = _tc_a(x_pad, W1, deg, deg).reshape(NC * NP, H)
    z1 = _agg_kernel(y1, src_aa, dst_t)
    y2 = _tc_b(z1, z1, y1, y1, deg, deg, b1r, W2).reshape(NC * NP, H)
    z2 = _agg_kernel(y2, src_aa, dst_t)
    h = _tc_c(z2, z2, y2, y2, deg, deg, b2r).reshape(NC * NP, H)
    return _edge_kernel(h, src_a, dst_a)


# speedup vs baseline: 1.0011x; 1.0011x over previous
import functools

import jax
import jax.numpy as jnp
from jax import lax
from jax.experimental import pallas as pl
from jax.experimental.pallas import tpu as pltpu
from jax.experimental.pallas import tpu_sc as plsc

N = 10000
NP = 10240
E = 160000
D = 256
H = 128
NC = 2
NS = 16
L = 16
NW = NC * NS

ET = E // NS
CE = 80
NCHUNK = ET // CE

STRIPE = NP // NS

_mesh = plsc.VectorSubcoreMesh(core_axis_name="c", subcore_axis_name="s",
                               num_cores=NC, num_subcores=NS)


def _zero_rows(buf, nrows):
    def _body(i, _):
        for j in range(H // L):
            buf[i, pl.ds(j * L, L)] = jnp.zeros((L,), jnp.float32)
        return _
    lax.fori_loop(0, nrows, _body, None)


_DEG_C = 40
_DEG_PER_TILE = (E // NC) // NS
_DEG_NCHUNK = _DEG_PER_TILE // _DEG_C
_DEG_G = 5


@functools.partial(
    pl.kernel,
    out_type=jax.ShapeDtypeStruct((NC * NP, H), jnp.float32),
    mesh=_mesh,
    scratch_types=[
        pltpu.VMEM((CE, H), jnp.float32),
        pltpu.VMEM((_DEG_NCHUNK, _DEG_C), jnp.int32),
        pltpu.VMEM_SHARED((NP, H), jnp.float32),
        pltpu.SemaphoreType.DMA,
    ],
)
def _deg_kernel(dstd_hbm, deg_out, zob, didx, hist_sh, sem):
    c = lax.axis_index("c")
    s = lax.axis_index("s")
    w = c * NS + s

    _zero_rows(zob, CE)
    row0 = s * STRIPE
    for j in range(STRIPE // CE):
        pltpu.sync_copy(zob, hist_sh.at[pl.ds(row0 + j * CE, CE)])

    pltpu.sync_copy(dstd_hbm.at[w], didx)

    def _ones_body(i, _):
        for j in range(H // L):
            zob[i, pl.ds(j * L, L)] = jnp.ones((L,), jnp.float32)
        return _
    lax.fori_loop(0, _DEG_C, _ones_body, None)
    plsc.subcore_barrier()

    ones_rows = zob.at[pl.ds(0, _DEG_C)]

    def _grp_body(g, _):
        for j in range(_DEG_G):
            k = g * _DEG_G + j
            pltpu.async_copy(ones_rows, hist_sh.at[didx.at[k]], sem,
                             add=True)
        for j in range(_DEG_G):
            pltpu.make_async_copy(ones_rows, hist_sh.at[didx.at[0]],
                                  sem).wait()
        return _
    lax.fori_loop(0, _DEG_NCHUNK // _DEG_G, _grp_body, None)

    plsc.subcore_barrier()
    pltpu.sync_copy(hist_sh.at[pl.ds(row0, STRIPE)],
                    deg_out.at[pl.ds(c * NP + row0, STRIPE)])


@functools.partial(
    pl.kernel,
    out_type=jax.ShapeDtypeStruct((NC * NP, H), jnp.float32),
    mesh=_mesh,
    scratch_types=[
        pltpu.VMEM((ET,), jnp.int32),
        pltpu.VMEM((NCHUNK, CE), jnp.int32),
        pltpu.VMEM((CE, H), jnp.float32),
        pltpu.VMEM((CE, H), jnp.float32),
        pltpu.VMEM_SHARED((NP, H), jnp.float32),
        pltpu.SemaphoreType.DMA,
        pltpu.SemaphoreType.DMA,
        pltpu.SemaphoreType.DMA,
        pltpu.SemaphoreType.DMA,
    ],
)
def _agg_kernel(y_hbm, srca_hbm, dstu_hbm, z_out, idx_s, idx_d,
                rows_a, rows_b, z_sh, sem_a, sem_b, ssem_a, ssem_b):
    c = lax.axis_index("c")
    s = lax.axis_index("s")
    w = c * NS + s

    _zero_rows(rows_a, CE)
    row0 = s * STRIPE
    for j in range(STRIPE // CE):
        pltpu.sync_copy(rows_a, z_sh.at[pl.ds(row0 + j * CE, CE)])

    pltpu.sync_copy(srca_hbm.at[w], idx_s)
    pltpu.sync_copy(dstu_hbm.at[s], idx_d)
    plsc.subcore_barrier()

    def _gather(k, buf, sem):
        pltpu.async_copy(y_hbm.at[idx_s.at[pl.ds(k * CE, CE)]], buf, sem)

    def _wait(buf, sem):
        pltpu.make_async_copy(y_hbm.at[pl.ds(0, CE)], buf, sem).wait()

    def _wait_scat(buf, ssem):
        pltpu.make_async_copy(buf, z_sh.at[idx_d.at[0]], ssem).wait()

    def _path(k, buf, gsem, ssem, obuf, ogsem, ossem):
        @pl.when(k >= 2)
        def _():
            _wait_scat(buf, ssem)
        _gather(k, buf, gsem)

        @pl.when(k >= 1)
        def _():
            _wait(obuf, ogsem)
            pltpu.async_copy(obuf, z_sh.at[idx_d.at[k - 1]], ossem,
                             add=True)

    def _chunk_body(k, _):
        @pl.when(k % 2 == 0)
        def _():
            _path(k, rows_a, sem_a, ssem_a, rows_b, sem_b, ssem_b)

        @pl.when(k % 2 == 1)
        def _():
            _path(k, rows_b, sem_b, ssem_b, rows_a, sem_a, ssem_a)
        return _
    lax.fori_loop(0, NCHUNK, _chunk_body, None)

    _wait(rows_a, sem_a)
    pltpu.sync_copy(rows_a, z_sh.at[idx_d.at[NCHUNK - 1]], add=True)
    _wait_scat(rows_b, ssem_b)

    plsc.subcore_barrier()
    pltpu.sync_copy(z_sh.at[pl.ds(row0, STRIPE)],
                    z_out.at[pl.ds(c * NP + row0, STRIPE)])


@functools.partial(
    pl.kernel,
    out_type=jax.ShapeDtypeStruct((E, D), jnp.float32),
    mesh=_mesh,
    scratch_types=[
        pltpu.VMEM((NCHUNK, CE), jnp.int32),
        pltpu.VMEM((NCHUNK, CE), jnp.int32),
        pltpu.VMEM((CE, H), jnp.float32),
        pltpu.VMEM((CE, H), jnp.float32),
        pltpu.SemaphoreType.DMA,
        pltpu.SemaphoreType.DMA,
        pltpu.SemaphoreType.DMA,
        pltpu.SemaphoreType.DMA,
    ],
)
def _edge_kernel(h_hbm, srca_hbm, dsta_hbm, out_hbm, idx_s, idx_d,
                 buf_a, buf_b, gsem_a, gsem_b, wsem_a, wsem_b):
    c = lax.axis_index("c")
    s = lax.axis_index("s")
    w = c * NS + s
    base = s * ET
    col0 = c * H

    pltpu.sync_copy(srca_hbm.at[w], idx_s)
    pltpu.sync_copy(dsta_hbm.at[w], idx_d)

    def _wait_write(buf, wsem):
        pltpu.make_async_copy(
            buf, out_hbm.at[pl.ds(base, CE), pl.ds(col0, H)], wsem).wait()

    def _wait_gather(buf, gsem):
        pltpu.make_async_copy(h_hbm.at[pl.ds(0, CE)], buf, gsem).wait()

    def _gsrc(k, buf, gsem):
        pltpu.async_copy(h_hbm.at[idx_s.at[k]], buf, gsem)

    _gsrc(0, buf_a, gsem_a)

    def _path(k, buf, gsem, wsem, obuf, ogsem, owsem):
        _wait_gather(buf, gsem)
        pltpu.async_copy(h_hbm.at[idx_d.at[k]], buf, gsem, add=True)

        @pl.when(k < NCHUNK - 1)
        def _():
            @pl.when(k >= 1)
            def _():
                _wait_write(obuf, owsem)
            _gsrc(k + 1, obuf, ogsem)

        _wait_gather(buf, gsem)
        off = pl.multiple_of(base + k * CE, 8)
        pltpu.async_copy(buf, out_hbm.at[pl.ds(off, CE), pl.ds(col0, H)],
                         wsem)

    def _chunk_body(k, _):
        @pl.when(k % 2 == 0)
        def _():
            _path(k, buf_a, gsem_a, wsem_a, buf_b, gsem_b, wsem_b)

        @pl.when(k % 2 == 1)
        def _():
            _path(k, buf_b, gsem_b, wsem_b, buf_a, gsem_a, wsem_a)
        return _
    lax.fori_loop(0, NCHUNK, _chunk_body, None)

    _wait_write(buf_a, wsem_a)
    _wait_write(buf_b, wsem_b)


_BR = 640


def _dinv_block(deg_a, deg_b):
    return lax.rsqrt(deg_a[:, 0:1] + deg_b[:, 0:1] + 1.0)


def _tc_mm_body(x_ref, w_ref, xw_ref):
    xw_ref[...] = jnp.dot(x_ref[...], w_ref[...],
                          preferred_element_type=jnp.float32)


def _tc_a_body(xw_ref, dega_ref, degb_ref, y_ref):
    dinv = _dinv_block(dega_ref[...], degb_ref[...])
    y = xw_ref[...] * dinv
    y_ref[0] = y[:, :H]
    y_ref[1] = y[:, H:]


def _tc_b_body(zlo_ref, zhi_ref, ylo_ref, yhi_ref, dega_ref, degb_ref,
               b1_ref, w2_ref, y2_ref):
    dinv = _dinv_block(dega_ref[...], degb_ref[...])
    h_lo = jnp.maximum((zlo_ref[...] + ylo_ref[...]) * dinv
                       + b1_ref[0:1, :H], 0.0)
    h_hi = jnp.maximum((zhi_ref[...] + yhi_ref[...]) * dinv
                       + b1_ref[0:1, H:], 0.0)
    xw2 = (jnp.dot(h_lo, w2_ref[:H, :], preferred_element_type=jnp.float32)
           + jnp.dot(h_hi, w2_ref[H:, :], preferred_element_type=jnp.float32))
    y2 = xw2 * dinv
    y2_ref[0] = y2[:, :H]
    y2_ref[1] = y2[:, H:]


def _tc_c_body(zlo_ref, zhi_ref, ylo_ref, yhi_ref, dega_ref, degb_ref,
               b2_ref, h_ref):
    dinv = _dinv_block(dega_ref[...], degb_ref[...])
    h_ref[0] = ((zlo_ref[...] + ylo_ref[...]) * dinv + b2_ref[0:1, :H]) * 0.5
    h_ref[1] = ((zhi_ref[...] + yhi_ref[...]) * dinv + b2_ref[0:1, H:]) * 0.5


_row_spec = pl.BlockSpec((_BR, D), lambda i: (i, 0))
_half_lo_spec = pl.BlockSpec((_BR, H), lambda i: (i, 0))
_half_hi_spec = pl.BlockSpec((_BR, H), lambda i: (NP // _BR + i, 0))
_w_spec = pl.BlockSpec((D, D), lambda i: (0, 0))
_b_spec = pl.BlockSpec((1, D), lambda i: (0, 0))
_split_out_spec = pl.BlockSpec((NC, _BR, H), lambda i: (0, i, 0))
_split_out_shape = jax.ShapeDtypeStruct((NC, NP, H), jnp.float32)

_tc_mm = pl.pallas_call(
    _tc_mm_body,
    grid=(NP // _BR,),
    in_specs=[_row_spec, _w_spec],
    out_specs=_row_spec,
    out_shape=jax.ShapeDtypeStruct((NP, D), jnp.float32),
)

_tc_a = pl.pallas_call(
    _tc_a_body,
    grid=(NP // _BR,),
    in_specs=[_row_spec, _half_lo_spec, _half_hi_spec],
    out_specs=_split_out_spec,
    out_shape=_split_out_shape,
)

_tc_b = pl.pallas_call(
    _tc_b_body,
    grid=(NP // _BR,),
    in_specs=[_half_lo_spec, _half_hi_spec, _half_lo_spec, _half_hi_spec,
              _half_lo_spec, _half_hi_spec, _b_spec, _w_spec],
    out_specs=_split_out_spec,
    out_shape=_split_out_shape,
)

_tc_c = pl.pallas_call(
    _tc_c_body,
    grid=(NP // _BR,),
    in_specs=[_half_lo_spec, _half_hi_spec, _half_lo_spec, _half_hi_spec,
              _half_lo_spec, _half_hi_spec, _b_spec],
    out_specs=_split_out_spec,
    out_shape=_split_out_shape,
)


def kernel(x, edge_index, W1, b1, W2, b2):
    ei = edge_index.astype(jnp.int32)
    src = ei[0]
    dst = ei[1]
    b1r = b1.reshape(1, D)
    b2r = b2.reshape(1, D)
    x_pad = jnp.concatenate(
        [x, jnp.zeros((NP - N, D), jnp.float32)], axis=0)

    src_t = src.reshape(NS, NCHUNK, CE)
    dst_t = dst.reshape(NS, NCHUNK, CE)
    src_a = jnp.concatenate([src_t, src_t + NP]).reshape(NW, NCHUNK, CE)
    dst_a = jnp.concatenate([dst_t, dst_t + NP]).reshape(NW, NCHUNK, CE)
    dst_d = dst.reshape(NW, _DEG_NCHUNK, _DEG_C)
    src_r = src.reshape(NS, ET)
    src_aa = jnp.concatenate([src_r, src_r + NP]).reshape(NW, ET)

    xw1 = _tc_mm(x_pad, W1)
    deg = _deg_kernel(dst_d)

    y1 = _tc_a(xw1, deg, deg).reshape(NC * NP, H)
    z1 = _agg_kernel(y1, src_aa, dst_t)
    y2 = _tc_b(z1, z1, y1, y1, deg, deg, b1r, W2).reshape(NC * NP, H)
    z2 = _agg_kernel(y2, src_aa, dst_t)
    h = _tc_c(z2, z2, y2, y2, deg, deg, b2r).reshape(NC * NP, H)
    return _edge_kernel(h, src_a, dst_a)
